# baseline (speedup 1.0000x reference)
import functools
import math

import jax
import jax.numpy as jnp
from jax import lax
from jax.experimental import pallas as pl
from jax.experimental.pallas import tpu as pltpu
from jax.experimental.pallas import tpu_sc as plsc

N, T, C, H2, K = 10000, 4, 128, 64, 8
TC_FLAT = T * C
N_PAD = 10240
RB = 256
CT = 2048
_INTERPRET = False


def _dense_body(x_ref, wmsg_ref, bmsg_ref, wself_ref, bself_ref,
                wproj_ref, sv_ref, sb_ref,
                msg_ref, selfp_ref, s_ref, e_ref, et_ref):
    xb = x_ref[...]
    msg = jax.lax.dot_general(
        xb, wmsg_ref[...], (((1,), (0,)), ((), ())),
        preferred_element_type=jnp.float32) + bmsg_ref[...]
    msg_ref[...] = msg
    sp = jax.lax.dot_general(
        xb, wself_ref[...], (((1,), (0,)), ((), ())),
        preferred_element_type=jnp.float32) + bself_ref[...]
    selfp_ref[...] = sp.reshape(RB, T, C).transpose(1, 0, 2)
    s = jax.lax.dot_general(
        msg, sv_ref[...], (((1,), (0,)), ((), ())),
        preferred_element_type=jnp.float32) + sb_ref[...]
    s_ref[...] = s.T
    e_un = jax.lax.dot_general(
        xb, wproj_ref[...], (((1,), (0,)), ((), ())),
        preferred_element_type=jnp.float32)
    nrm = jnp.sqrt(jnp.sum(e_un * e_un, axis=1, keepdims=True))
    e = e_un / (nrm + 1e-12)
    e_ref[...] = e
    et_ref[...] = e.T


def _dense_call(x_flat, wmsg, bmsg, wself, bself, wproj, sv, sb):
    grid = (N_PAD // RB,)
    return pl.pallas_call(
        _dense_body,
        grid=grid,
        in_specs=[
            pl.BlockSpec((RB, TC_FLAT), lambda i: (i, 0)),
            pl.BlockSpec((TC_FLAT, TC_FLAT), lambda i: (0, 0)),
            pl.BlockSpec((1, TC_FLAT), lambda i: (0, 0)),
            pl.BlockSpec((TC_FLAT, TC_FLAT), lambda i: (0, 0)),
            pl.BlockSpec((1, TC_FLAT), lambda i: (0, 0)),
            pl.BlockSpec((TC_FLAT, H2), lambda i: (0, 0)),
            pl.BlockSpec((TC_FLAT, 8), lambda i: (0, 0)),
            pl.BlockSpec((1, 8), lambda i: (0, 0)),
        ],
        out_specs=[
            pl.BlockSpec((RB, TC_FLAT), lambda i: (i, 0)),
            pl.BlockSpec((T, RB, C), lambda i: (0, i, 0)),
            pl.BlockSpec((8, RB), lambda i: (0, i)),
            pl.BlockSpec((RB, H2), lambda i: (i, 0)),
            pl.BlockSpec((H2, RB), lambda i: (0, i)),
        ],
        out_shape=[
            jax.ShapeDtypeStruct((N_PAD, TC_FLAT), jnp.float32),
            jax.ShapeDtypeStruct((T, N_PAD, C), jnp.float32),
            jax.ShapeDtypeStruct((8, N_PAD), jnp.float32),
            jax.ShapeDtypeStruct((N_PAD, H2), jnp.float32),
            jax.ShapeDtypeStruct((H2, N_PAD), jnp.float32),
        ],
        interpret=_INTERPRET,
    )(x_flat, wmsg, bmsg, wself, bself, wproj, sv, sb)


def _topk_body(e_ref, et_ref, tv_ref, ti_ref):
    i = pl.program_id(0)
    er = e_ref[...]
    row_g = i * RB + jax.lax.broadcasted_iota(jnp.int32, (RB, 1), 0)
    run_v = jnp.full((RB, K), -jnp.inf, jnp.float32)
    run_i = jnp.zeros((RB, K), jnp.int32)
    pos16 = jax.lax.broadcasted_iota(jnp.int32, (RB, 2 * K), 1)
    for ct in range(N_PAD // CT):
        sim = jax.lax.dot_general(
            er, et_ref[:, ct * CT:(ct + 1) * CT], (((1,), (0,)), ((), ())),
            preferred_element_type=jnp.float32)
        colg = ct * CT + jax.lax.broadcasted_iota(jnp.int32, (RB, CT), 1)
        sim = jnp.where((colg == row_g) | (colg >= N), -jnp.inf, sim)
        tvals, tidx = [], []
        for _ in range(K):
            m = jnp.max(sim, axis=1, keepdims=True)
            cand = jnp.where(sim == m, colg, jnp.int32(2 ** 30))
            am = jnp.min(cand, axis=1, keepdims=True)
            sim = jnp.where(colg == am, -jnp.inf, sim)
            tvals.append(m)
            tidx.append(am)
        cv = jnp.concatenate([run_v] + tvals, axis=1)
        ci = jnp.concatenate([run_i] + tidx, axis=1)
        nv, ni = [], []
        for _ in range(K):
            m = jnp.max(cv, axis=1, keepdims=True)
            p = jnp.where(cv == m, pos16, jnp.int32(2 ** 30))
            pm = jnp.min(p, axis=1, keepdims=True)
            sel = pos16 == pm
            ni.append(jnp.max(jnp.where(sel, ci, -1), axis=1, keepdims=True))
            nv.append(m)
            cv = jnp.where(sel, -jnp.inf, cv)
        run_v = jnp.concatenate(nv, axis=1)
        run_i = jnp.concatenate(ni, axis=1)
    tv_ref[...] = run_v
    ti_ref[...] = run_i


def _topk_call(e, et):
    grid = (N_PAD // RB,)
    return pl.pallas_call(
        _topk_body,
        grid=grid,
        in_specs=[
            pl.BlockSpec((RB, H2), lambda i: (i, 0)),
            pl.BlockSpec((H2, N_PAD), lambda i: (0, 0)),
        ],
        out_specs=[
            pl.BlockSpec((RB, K), lambda i: (i, 0)),
            pl.BlockSpec((RB, K), lambda i: (i, 0)),
        ],
        out_shape=[
            jax.ShapeDtypeStruct((N_PAD, K), jnp.float32),
            jax.ShapeDtypeStruct((N_PAD, K), jnp.int32),
        ],
        interpret=_INTERPRET,
    )(e, et)


EP = 245760
E_ROWS = EP // 128
BLK_E = 2048
NBLK_E = EP // BLK_E
SB = 128
NPT = N_PAD // 16
FCH = 128


def _edge_call(msg4, sT, selfp_t, pk, zrow):
    mesh = plsc.VectorSubcoreMesh(core_axis_name="c", subcore_axis_name="s")

    @functools.partial(
        pl.kernel,
        mesh=mesh,
        compiler_params=pltpu.CompilerParams(needs_layout_passes=False),
        out_type=jax.ShapeDtypeStruct((T, N_PAD, C), jnp.float32),
        scratch_types=[
            pltpu.VMEM((NPT, C), jnp.float32),
            pltpu.VMEM((NPT,), jnp.float32),
            pltpu.VMEM((N_PAD,), jnp.float32),
            pltpu.VMEM((NPT,), jnp.float32),
            pltpu.VMEM((48, 128), jnp.int32),
            pltpu.VMEM((BLK_E + 16,), jnp.int32),
            pltpu.VMEM((SB, C), jnp.float32),
            pltpu.VMEM((SB,), jnp.float32),
            pltpu.VMEM((SB,), jnp.float32),
            pltpu.VMEM((SB,), jnp.int32),
            pltpu.VMEM((SB,), jnp.int32),
            pltpu.VMEM((FCH,), jnp.float32),
            pltpu.SemaphoreType.DMA,
        ],
    )
    def k(msg4_h, sT_h, selfp_h, pk_h, zrow_h, out_h,
          agg, den, ssrc_t, sdst_my, ebuf, posb, rows, wtb, exb, dlb, midx,
          invb, gsem):
        cc = lax.axis_index("c")
        ss = lax.axis_index("s")
        iota16 = lax.broadcasted_iota(jnp.int32, (16,), 0)
        z16 = jnp.zeros((16,), jnp.int32)
        zf16 = jnp.zeros((16,), jnp.float32)
        n_lo = ss * NPT
        for i in range((BLK_E + 16) // 16):
            posb[pl.ds(i * 16, 16)] = z16
        for tp in range(2):
            t = 2 * cc + tp
            pltpu.sync_copy(zrow_h, agg)
            for i in range(NPT // 16):
                den[pl.ds(i * 16, 16)] = zf16
            pltpu.sync_copy(sT_h.at[t], ssrc_t)
            pltpu.sync_copy(sT_h.at[t + T, pl.ds(n_lo, NPT)], sdst_my)

            def blk_body(b, carry):
                pltpu.sync_copy(pk_h.at[pl.ds(b * 48, 48)], ebuf)

                def cgroup(g, cnt):
                    dv = plsc.load_gather(
                        ebuf, [z16 + (16 + (g >> 3)),
                               iota16 + ((g & 7) * 16)])
                    dl = dv - n_lo
                    own = (dl >= 0) & (dl < NPT)
                    p16 = g * 16 + iota16
                    plsc.store_compressed(posb.at[pl.ds(cnt, 16)], p16,
                                          mask=own)
                    c16 = plsc.all_reduce_population_count(own)
                    return cnt + jnp.max(c16)
                m_tot = lax.fori_loop(0, BLK_E // 16, cgroup, jnp.int32(0))
                nsb = (m_tot + (SB - 1)) // SB

                def sb_body(sb, c2):
                    base = sb * SB
                    mcnt = jnp.minimum(m_tot - base, SB)
                    for v in range(SB // 16):
                        p16 = plsc.load_gather(posb,
                                               [iota16 + (base + v * 16)])
                        p16 = p16 & (BLK_E - 1)
                        grow = lax.shift_right_logical(p16, 7)
                        lane = p16 & 127
                        sv = plsc.load_gather(ebuf, [grow, lane])
                        dv = plsc.load_gather(ebuf, [grow + 16, lane])
                        eav16 = plsc.bitcast(
                            plsc.load_gather(ebuf, [grow + 32, lane]),
                            jnp.float32)
                        dls = jnp.minimum(jnp.maximum(dv - n_lo, 0), NPT - 1)
                        a = (plsc.load_gather(ssrc_t, [sv])
                             + plsc.load_gather(sdst_my, [dls]))
                        a = jnp.maximum(a, 0.2 * a)
                        ex = jnp.exp(a)
                        sl = pl.ds(v * 16, 16)
                        exb[sl] = ex
                        wtb[sl] = ex * eav16
                        dlb[sl] = dls
                        midx[sl] = sv * T + t
                    pltpu.async_copy(msg4_h.at[midx], rows, gsem).wait()

                    def e_body(je, c3):
                        dlv = plsc.load_gather(dlb, [z16 + je])
                        w16 = plsc.load_gather(wtb, [z16 + je])
                        e16 = plsc.load_gather(exb, [z16 + je])
                        cur = plsc.load_gather(den, [dlv])
                        plsc.store_scatter(den, [dlv], cur + e16)
                        dl0 = jnp.max(dlv)
                        for kk in range(C // 16):
                            sl2 = pl.ds(kk * 16, 16)
                            agg[dl0, sl2] = (agg[dl0, sl2]
                                             + rows[je, sl2] * w16)
                        return c3
                    lax.fori_loop(0, mcnt, e_body, 0)
                    return c2
                lax.fori_loop(0, nsb, sb_body, 0)
                return carry

            lax.fori_loop(0, NBLK_E, blk_body, 0)
            for ck in range(NPT // FCH):
                nbl = ck * FCH
                nbg = n_lo + nbl
                pltpu.sync_copy(selfp_h.at[t, pl.ds(nbg, FCH)], rows)
                for v in range(FCH // 16):
                    invb[pl.ds(v * 16, 16)] = 1.0 / (
                        den[pl.ds(nbl + v * 16, 16)] + 1e-16)

                def n_body(jn, c2):
                    w16 = plsc.load_gather(invb, [z16 + jn])
                    for kk in range(C // 16):
                        sl2 = pl.ds(kk * 16, 16)
                        rows[jn, sl2] = (agg[nbl + jn, sl2] * w16
                                         + rows[jn, sl2])
                    return c2
                lax.fori_loop(0, FCH, n_body, 0)
                pltpu.sync_copy(rows, out_h.at[t, pl.ds(nbg, FCH)])

    return k(msg4, sT, selfp_t, pk, zrow)


def kernel(x, edge_index, edge_attr, node_proj_W, mix_logit, conv_W, conv_b,
           lin_msg_W, lin_msg_b, lin_self_W, lin_self_b,
           att_src_W, att_src_b, att_dst_W, att_dst_b):
    f32 = jnp.float32
    x_flat = x.reshape(N, TC_FLAT)
    x_flat = jnp.pad(x_flat, ((0, N_PAD - N), (0, 0)))

    eyeT = jnp.eye(T, dtype=f32)
    blocks = []
    for t_in in range(T):
        row = []
        for t_out in range(T):
            k = t_in - t_out + 1
            if 0 <= k <= 2:
                row.append(conv_W[:, :, k].T)
            else:
                row.append(jnp.zeros((C, C), f32))
        blocks.append(jnp.concatenate(row, axis=1))
    wconv = jnp.concatenate(blocks, axis=0)
    wm_bd = jnp.kron(eyeT, lin_msg_W.T)
    wmsg = wconv @ wm_bd
    bmsg_t = conv_b @ lin_msg_W.T + lin_msg_b
    bmsg = jnp.tile(bmsg_t, (T,))[None, :]
    wself = jnp.kron(eyeT, lin_self_W.T)
    bself = jnp.tile(lin_self_b, (T,))[None, :]
    wproj = jnp.tile(node_proj_W.T, (T, 1)) / T
    sv = jnp.zeros((TC_FLAT, 2 * T), f32)
    for t in range(T):
        sv = sv.at[t * C:(t + 1) * C, t].set(att_src_W[0])
        sv = sv.at[t * C:(t + 1) * C, T + t].set(att_dst_W[0])
    sb = jnp.concatenate([jnp.tile(att_src_b, (T,)),
                          jnp.tile(att_dst_b, (T,))])[None, :]

    msg_flat, selfp_t, sT, e, et = _dense_call(
        x_flat, wmsg, bmsg, wself, bself, wproj, sv, sb)
    tv_p, ti_p = _topk_call(e, et)
    tv = tv_p[:N]
    ti = ti_p[:N]

    alpha = jax.nn.sigmoid(mix_logit)
    e_fixed = edge_index.shape[1]
    n_pad_e = EP - (e_fixed + N * K)
    src_dyn = jnp.arange(N * K, dtype=jnp.int32) // K
    dst_dyn = ti.reshape(-1)
    srcs = jnp.concatenate([edge_index[0].astype(jnp.int32), src_dyn,
                            jnp.zeros((n_pad_e,), jnp.int32)])
    dsts = jnp.concatenate([edge_index[1].astype(jnp.int32), dst_dyn,
                            jnp.full((n_pad_e,), -1, jnp.int32)])
    eav = jnp.concatenate([edge_attr[:, 0] * (1.0 - alpha),
                           tv.reshape(-1) * alpha,
                           jnp.zeros((n_pad_e,), f32)])
    pk = jnp.concatenate(
        [srcs.reshape(NBLK_E, 16, 128),
         dsts.reshape(NBLK_E, 16, 128),
         jax.lax.bitcast_convert_type(eav, jnp.int32).reshape(NBLK_E, 16, 128)],
        axis=1).reshape(NBLK_E * 48, 128)
    msg4 = msg_flat.reshape(N_PAD * T, C)
    zrow = jnp.zeros((NPT, C), f32)
    out_t = _edge_call(msg4, sT, selfp_t, pk, zrow)
    return out_t.transpose(1, 0, 2)[:N]


# speedup vs baseline: 1.3829x; 1.3829x over previous
import functools
import math

import jax
import jax.numpy as jnp
from jax import lax
from jax.experimental import pallas as pl
from jax.experimental.pallas import tpu as pltpu
from jax.experimental.pallas import tpu_sc as plsc

N, T, C, H2, K = 10000, 4, 128, 64, 8
TC_FLAT = T * C
N_PAD = 10240
RB = 256
CT = 2048
_INTERPRET = False


def _dense_body(x_ref, wmsg_ref, bmsg_ref, wself_ref, bself_ref,
                wproj_ref, sv_ref, sb_ref,
                msg_ref, selfp_ref, s_ref, e_ref, et_ref):
    xb = x_ref[...]
    msg = jax.lax.dot_general(
        xb, wmsg_ref[...], (((1,), (0,)), ((), ())),
        preferred_element_type=jnp.float32) + bmsg_ref[...]
    msg_ref[...] = msg
    sp = jax.lax.dot_general(
        xb, wself_ref[...], (((1,), (0,)), ((), ())),
        preferred_element_type=jnp.float32) + bself_ref[...]
    selfp_ref[...] = sp.reshape(RB, T, C).transpose(1, 0, 2)
    s = jax.lax.dot_general(
        msg, sv_ref[...], (((1,), (0,)), ((), ())),
        preferred_element_type=jnp.float32) + sb_ref[...]
    s_ref[...] = s.T
    e_un = jax.lax.dot_general(
        xb, wproj_ref[...], (((1,), (0,)), ((), ())),
        preferred_element_type=jnp.float32)
    nrm = jnp.sqrt(jnp.sum(e_un * e_un, axis=1, keepdims=True))
    e = e_un / (nrm + 1e-12)
    e_ref[...] = e
    et_ref[...] = e.T


def _dense_call(x_flat, wmsg, bmsg, wself, bself, wproj, sv, sb):
    grid = (N_PAD // RB,)
    return pl.pallas_call(
        _dense_body,
        grid=grid,
        in_specs=[
            pl.BlockSpec((RB, TC_FLAT), lambda i: (i, 0)),
            pl.BlockSpec((TC_FLAT, TC_FLAT), lambda i: (0, 0)),
            pl.BlockSpec((1, TC_FLAT), lambda i: (0, 0)),
            pl.BlockSpec((TC_FLAT, TC_FLAT), lambda i: (0, 0)),
            pl.BlockSpec((1, TC_FLAT), lambda i: (0, 0)),
            pl.BlockSpec((TC_FLAT, H2), lambda i: (0, 0)),
            pl.BlockSpec((TC_FLAT, 8), lambda i: (0, 0)),
            pl.BlockSpec((1, 8), lambda i: (0, 0)),
        ],
        out_specs=[
            pl.BlockSpec((RB, TC_FLAT), lambda i: (i, 0)),
            pl.BlockSpec((T, RB, C), lambda i: (0, i, 0)),
            pl.BlockSpec((8, RB), lambda i: (0, i)),
            pl.BlockSpec((RB, H2), lambda i: (i, 0)),
            pl.BlockSpec((H2, RB), lambda i: (0, i)),
        ],
        out_shape=[
            jax.ShapeDtypeStruct((N_PAD, TC_FLAT), jnp.float32),
            jax.ShapeDtypeStruct((T, N_PAD, C), jnp.float32),
            jax.ShapeDtypeStruct((8, N_PAD), jnp.float32),
            jax.ShapeDtypeStruct((N_PAD, H2), jnp.float32),
            jax.ShapeDtypeStruct((H2, N_PAD), jnp.float32),
        ],
        interpret=_INTERPRET,
    )(x_flat, wmsg, bmsg, wself, bself, wproj, sv, sb)


def _topk_body(e_ref, et_ref, tv_ref, ti_ref):
    i = pl.program_id(0)
    er = e_ref[...]
    row_g = i * RB + jax.lax.broadcasted_iota(jnp.int32, (RB, 1), 0)
    run_v = jnp.full((RB, K), -jnp.inf, jnp.float32)
    run_i = jnp.zeros((RB, K), jnp.int32)
    pos16 = jax.lax.broadcasted_iota(jnp.int32, (RB, 2 * K), 1)
    for ct in range(N_PAD // CT):
        sim = jax.lax.dot_general(
            er, et_ref[:, ct * CT:(ct + 1) * CT], (((1,), (0,)), ((), ())),
            preferred_element_type=jnp.float32)
        colg = ct * CT + jax.lax.broadcasted_iota(jnp.int32, (RB, CT), 1)
        sim = jnp.where((colg == row_g) | (colg >= N), -jnp.inf, sim)
        tvals, tidx = [], []
        for _ in range(K):
            m = jnp.max(sim, axis=1, keepdims=True)
            cand = jnp.where(sim == m, colg, jnp.int32(2 ** 30))
            am = jnp.min(cand, axis=1, keepdims=True)
            sim = jnp.where(colg == am, -jnp.inf, sim)
            tvals.append(m)
            tidx.append(am)
        cv = jnp.concatenate([run_v] + tvals, axis=1)
        ci = jnp.concatenate([run_i] + tidx, axis=1)
        nv, ni = [], []
        for _ in range(K):
            m = jnp.max(cv, axis=1, keepdims=True)
            p = jnp.where(cv == m, pos16, jnp.int32(2 ** 30))
            pm = jnp.min(p, axis=1, keepdims=True)
            sel = pos16 == pm
            ni.append(jnp.max(jnp.where(sel, ci, -1), axis=1, keepdims=True))
            nv.append(m)
            cv = jnp.where(sel, -jnp.inf, cv)
        run_v = jnp.concatenate(nv, axis=1)
        run_i = jnp.concatenate(ni, axis=1)
    tv_ref[...] = run_v
    ti_ref[...] = run_i


def _topk_call(e, et):
    grid = (N_PAD // RB,)
    return pl.pallas_call(
        _topk_body,
        grid=grid,
        in_specs=[
            pl.BlockSpec((RB, H2), lambda i: (i, 0)),
            pl.BlockSpec((H2, N_PAD), lambda i: (0, 0)),
        ],
        out_specs=[
            pl.BlockSpec((RB, K), lambda i: (i, 0)),
            pl.BlockSpec((RB, K), lambda i: (i, 0)),
        ],
        out_shape=[
            jax.ShapeDtypeStruct((N_PAD, K), jnp.float32),
            jax.ShapeDtypeStruct((N_PAD, K), jnp.int32),
        ],
        interpret=_INTERPRET,
    )(e, et)


EP = 245760
BLK = 64
NBLK = EP // 16 // BLK
NPT = N_PAD // 16
FCH = 32


def _edge_call(msg4, sT, selfp_t, pk, zrow, zd):
    mesh = plsc.VectorSubcoreMesh(core_axis_name="c", subcore_axis_name="s")

    @functools.partial(
        pl.kernel,
        mesh=mesh,
        compiler_params=pltpu.CompilerParams(needs_layout_passes=False),
        out_type=jax.ShapeDtypeStruct((T, N_PAD, C), jnp.float32),
        scratch_types=[
            pltpu.VMEM_SHARED((N_PAD, C), jnp.float32),
            pltpu.VMEM_SHARED((N_PAD,), jnp.float32),
            pltpu.VMEM((BLK, C), jnp.float32),
            pltpu.VMEM((BLK, C), jnp.float32),
            pltpu.VMEM((N_PAD,), jnp.float32),
            pltpu.VMEM((N_PAD,), jnp.float32),
            pltpu.VMEM((3 * BLK,), jnp.int32),
            pltpu.VMEM((3 * BLK,), jnp.int32),
            pltpu.VMEM((BLK,), jnp.int32),
            pltpu.VMEM((BLK,), jnp.int32),
            pltpu.VMEM((BLK,), jnp.float32),
            pltpu.VMEM((BLK,), jnp.float32),
            pltpu.VMEM((BLK,), jnp.int32),
            pltpu.VMEM((FCH,), jnp.float32),
            pltpu.SemaphoreType.DMA,
            pltpu.SemaphoreType.DMA,
            pltpu.SemaphoreType.DMA,
        ],
    )
    def k(msg4_h, sT_h, selfp_h, pk_h, zrow_h, zd_h, out_h,
          agg_sh, den_sh, rows0, rows1, ssrc_t, sdst_t, pkb0, pkb1,
          dst0, dst1, ex_b, wt_b, mix_b, inv_b, gsem, ssem0, ssem1):
        cc = lax.axis_index("c")
        ss = lax.axis_index("s")
        iota16 = lax.broadcasted_iota(jnp.int32, (16,), 0)
        z16 = jnp.zeros((16,), jnp.int32)
        n0 = ss * NPT
        blk0 = ss * NBLK
        bufs = ((rows0, dst0, pkb0, ssem0), (rows1, dst1, pkb1, ssem1))
        for tp in range(2):
            t = 2 * cc + tp
            pltpu.sync_copy(zrow_h, agg_sh.at[pl.ds(n0, NPT)])
            pltpu.sync_copy(zd_h, den_sh.at[pl.ds(n0, NPT)])
            pltpu.sync_copy(sT_h.at[t], ssrc_t)
            pltpu.sync_copy(sT_h.at[t + T], sdst_t)
            plsc.subcore_barrier()

            def pair_body(ob, carry):
                for p in range(2):
                    rowsP, dstP, pkbP, ssemP = bufs[p]
                    bb = ob * 2 + p

                    @pl.when(ob > 0)
                    def _drain():
                        pltpu.make_async_copy(
                            rowsP, agg_sh.at[dstP], ssemP).wait()

                    pltpu.sync_copy(
                        pk_h.at[pl.ds((blk0 + bb) * (3 * BLK), 3 * BLK)],
                        pkbP)
                    for j in range(BLK // 16):
                        sl = pl.ds(j * 16, 16)
                        sv = pkbP[sl]
                        dv = pkbP[pl.ds(BLK + j * 16, 16)]
                        eav16 = plsc.bitcast(
                            pkbP[pl.ds(2 * BLK + j * 16, 16)], jnp.float32)
                        a = (plsc.load_gather(ssrc_t, [sv])
                             + plsc.load_gather(sdst_t, [dv]))
                        a = jnp.maximum(a, 0.2 * a)
                        ex = jnp.exp(a)
                        ex_b[sl] = ex
                        wt_b[sl] = ex * eav16
                        dstP[sl] = dv
                        mix_b[sl] = sv * T + t
                    pltpu.sync_copy(ex_b, den_sh.at[dstP], add=True)
                    pltpu.async_copy(msg4_h.at[mix_b], rowsP, gsem).wait()

                    def e_body(ie, c2):
                        w16 = plsc.load_gather(wt_b, [z16 + ie])
                        rid = z16 + ie
                        for kk in range(C // 16):
                            col = iota16 + (kk * 16)
                            v = plsc.load_gather(rowsP, [rid, col])
                            plsc.store_scatter(rowsP, [rid, col], v * w16)
                        return c2
                    lax.fori_loop(0, BLK, e_body, 0)
                    pltpu.async_copy(rowsP, agg_sh.at[dstP], ssemP, add=True)
                return carry

            lax.fori_loop(0, NBLK // 2, pair_body, 0)
            pltpu.make_async_copy(rows0, agg_sh.at[dst0], ssem0).wait()
            pltpu.make_async_copy(rows1, agg_sh.at[dst1], ssem1).wait()
            plsc.subcore_barrier()
            for ck in range(NPT // FCH):
                nb = n0 + ck * FCH
                pltpu.sync_copy(den_sh.at[pl.ds(nb, FCH)], inv_b)
                for j in range(FCH // 16):
                    sl = pl.ds(j * 16, 16)
                    inv_b[sl] = 1.0 / (inv_b[sl] + 1e-16)
                pltpu.sync_copy(agg_sh.at[pl.ds(nb, FCH)],
                                rows0.at[pl.ds(0, FCH)])
                pltpu.sync_copy(selfp_h.at[t, pl.ds(nb, FCH)],
                                rows0.at[pl.ds(FCH, FCH)])

                def n_body(jn, c2):
                    w16 = plsc.load_gather(inv_b, [z16 + jn])
                    for kk in range(C // 16):
                        col = iota16 + (kk * 16)
                        v = plsc.load_gather(rows0, [z16 + jn, col])
                        sv_ = plsc.load_gather(rows0, [z16 + (FCH + jn), col])
                        plsc.store_scatter(rows1, [z16 + jn, col],
                                           v * w16 + sv_)
                    return c2
                lax.fori_loop(0, FCH, n_body, 0)
                pltpu.sync_copy(rows1.at[pl.ds(0, FCH)],
                                out_h.at[t, pl.ds(nb, FCH)])
            plsc.subcore_barrier()

    return k(msg4, sT, selfp_t, pk, zrow, zd)


def kernel(x, edge_index, edge_attr, node_proj_W, mix_logit, conv_W, conv_b,
           lin_msg_W, lin_msg_b, lin_self_W, lin_self_b,
           att_src_W, att_src_b, att_dst_W, att_dst_b):
    f32 = jnp.float32
    x_flat = x.reshape(N, TC_FLAT)
    x_flat = jnp.pad(x_flat, ((0, N_PAD - N), (0, 0)))

    eyeT = jnp.eye(T, dtype=f32)
    blocks = []
    for t_in in range(T):
        row = []
        for t_out in range(T):
            k = t_in - t_out + 1
            if 0 <= k <= 2:
                row.append(conv_W[:, :, k].T)
            else:
                row.append(jnp.zeros((C, C), f32))
        blocks.append(jnp.concatenate(row, axis=1))
    wconv = jnp.concatenate(blocks, axis=0)
    wm_bd = jnp.kron(eyeT, lin_msg_W.T)
    wmsg = wconv @ wm_bd
    bmsg_t = conv_b @ lin_msg_W.T + lin_msg_b
    bmsg = jnp.tile(bmsg_t, (T,))[None, :]
    wself = jnp.kron(eyeT, lin_self_W.T)
    bself = jnp.tile(lin_self_b, (T,))[None, :]
    wproj = jnp.tile(node_proj_W.T, (T, 1)) / T
    sv = jnp.zeros((TC_FLAT, 2 * T), f32)
    for t in range(T):
        sv = sv.at[t * C:(t + 1) * C, t].set(att_src_W[0])
        sv = sv.at[t * C:(t + 1) * C, T + t].set(att_dst_W[0])
    sb = jnp.concatenate([jnp.tile(att_src_b, (T,)),
                          jnp.tile(att_dst_b, (T,))])[None, :]

    msg_flat, selfp_t, sT, e, et = _dense_call(
        x_flat, wmsg, bmsg, wself, bself, wproj, sv, sb)
    tv_p, ti_p = _topk_call(e, et)
    tv = tv_p[:N]
    ti = ti_p[:N]

    alpha = jax.nn.sigmoid(mix_logit)
    e_fixed = edge_index.shape[1]
    n_pad_e = EP - (e_fixed + N * K)
    src_dyn = jnp.arange(N * K, dtype=jnp.int32) // K
    dst_dyn = ti.reshape(-1)
    pad_dst = N + (jnp.arange(n_pad_e, dtype=jnp.int32) % (N_PAD - N))
    srcs = jnp.concatenate([edge_index[0].astype(jnp.int32), src_dyn,
                            jnp.zeros((n_pad_e,), jnp.int32)])
    dsts = jnp.concatenate([edge_index[1].astype(jnp.int32), dst_dyn, pad_dst])
    eav = jnp.concatenate([edge_attr[:, 0] * (1.0 - alpha),
                           tv.reshape(-1) * alpha,
                           jnp.zeros((n_pad_e,), f32)])
    pk = jnp.concatenate(
        [srcs.reshape(-1, BLK), dsts.reshape(-1, BLK),
         jax.lax.bitcast_convert_type(eav, jnp.int32).reshape(-1, BLK)],
        axis=1).reshape(-1)
    msg4 = msg_flat.reshape(N_PAD * T, C)
    zrow = jnp.zeros((NPT, C), f32)
    zd = jnp.zeros((NPT,), f32)
    out_t = _edge_call(msg4, sT, selfp_t, pk, zrow, zd)
    return out_t.transpose(1, 0, 2)[:N]


# speedup vs baseline: 2.2041x; 1.5938x over previous
import functools
import math

import jax
import jax.numpy as jnp
from jax import lax
from jax.experimental import pallas as pl
from jax.experimental.pallas import tpu as pltpu
from jax.experimental.pallas import tpu_sc as plsc

N, T, C, H2, K = 10000, 4, 128, 64, 8
TC_FLAT = T * C
N_PAD = 10240
RB = 256
CT = 2048
_INTERPRET = False


def _dense_body(x_ref, wmsg_ref, bmsg_ref, wself_ref, bself_ref,
                wproj_ref, sv_ref, sb_ref,
                msg_ref, selfp_ref, s_ref, e_ref, et_ref):
    xb = x_ref[...]
    msg = jax.lax.dot_general(
        xb, wmsg_ref[...], (((1,), (0,)), ((), ())),
        preferred_element_type=jnp.float32) + bmsg_ref[...]
    msg_ref[...] = msg
    sp = jax.lax.dot_general(
        xb, wself_ref[...], (((1,), (0,)), ((), ())),
        preferred_element_type=jnp.float32) + bself_ref[...]
    selfp_ref[...] = sp.reshape(RB, T, C).transpose(1, 0, 2)
    s = jax.lax.dot_general(
        msg, sv_ref[...], (((1,), (0,)), ((), ())),
        preferred_element_type=jnp.float32) + sb_ref[...]
    s_ref[...] = s.T
    e_un = jax.lax.dot_general(
        xb, wproj_ref[...], (((1,), (0,)), ((), ())),
        preferred_element_type=jnp.float32)
    nrm = jnp.sqrt(jnp.sum(e_un * e_un, axis=1, keepdims=True))
    e = e_un / (nrm + 1e-12)
    e_ref[...] = e
    et_ref[...] = e.T


def _dense_call(x_flat, wmsg, bmsg, wself, bself, wproj, sv, sb):
    grid = (N_PAD // RB,)
    return pl.pallas_call(
        _dense_body,
        grid=grid,
        in_specs=[
            pl.BlockSpec((RB, TC_FLAT), lambda i: (i, 0)),
            pl.BlockSpec((TC_FLAT, TC_FLAT), lambda i: (0, 0)),
            pl.BlockSpec((1, TC_FLAT), lambda i: (0, 0)),
            pl.BlockSpec((TC_FLAT, TC_FLAT), lambda i: (0, 0)),
            pl.BlockSpec((1, TC_FLAT), lambda i: (0, 0)),
            pl.BlockSpec((TC_FLAT, H2), lambda i: (0, 0)),
            pl.BlockSpec((TC_FLAT, 8), lambda i: (0, 0)),
            pl.BlockSpec((1, 8), lambda i: (0, 0)),
        ],
        out_specs=[
            pl.BlockSpec((RB, TC_FLAT), lambda i: (i, 0)),
            pl.BlockSpec((T, RB, C), lambda i: (0, i, 0)),
            pl.BlockSpec((8, RB), lambda i: (0, i)),
            pl.BlockSpec((RB, H2), lambda i: (i, 0)),
            pl.BlockSpec((H2, RB), lambda i: (0, i)),
        ],
        out_shape=[
            jax.ShapeDtypeStruct((N_PAD, TC_FLAT), jnp.float32),
            jax.ShapeDtypeStruct((T, N_PAD, C), jnp.float32),
            jax.ShapeDtypeStruct((8, N_PAD), jnp.float32),
            jax.ShapeDtypeStruct((N_PAD, H2), jnp.float32),
            jax.ShapeDtypeStruct((H2, N_PAD), jnp.float32),
        ],
        interpret=_INTERPRET,
    )(x_flat, wmsg, bmsg, wself, bself, wproj, sv, sb)


def _topk_body(e_ref, et_ref, tv_ref, ti_ref):
    i = pl.program_id(0)
    er = e_ref[...]
    row_g = i * RB + jax.lax.broadcasted_iota(jnp.int32, (RB, 1), 0)
    run_v = jnp.full((RB, K), -jnp.inf, jnp.float32)
    run_i = jnp.zeros((RB, K), jnp.int32)
    pos16 = jax.lax.broadcasted_iota(jnp.int32, (RB, 2 * K), 1)
    for ct in range(N_PAD // CT):
        sim = jax.lax.dot_general(
            er, et_ref[:, ct * CT:(ct + 1) * CT], (((1,), (0,)), ((), ())),
            preferred_element_type=jnp.float32)
        colg = ct * CT + jax.lax.broadcasted_iota(jnp.int32, (RB, CT), 1)
        sim = jnp.where((colg == row_g) | (colg >= N), -jnp.inf, sim)
        tvals, tidx = [], []
        for _ in range(K):
            m = jnp.max(sim, axis=1, keepdims=True)
            cand = jnp.where(sim == m, colg, jnp.int32(2 ** 30))
            am = jnp.min(cand, axis=1, keepdims=True)
            sim = jnp.where(colg == am, -jnp.inf, sim)
            tvals.append(m)
            tidx.append(am)
        cv = jnp.concatenate([run_v] + tvals, axis=1)
        ci = jnp.concatenate([run_i] + tidx, axis=1)
        nv, ni = [], []
        for _ in range(K):
            m = jnp.max(cv, axis=1, keepdims=True)
            p = jnp.where(cv == m, pos16, jnp.int32(2 ** 30))
            pm = jnp.min(p, axis=1, keepdims=True)
            sel = pos16 == pm
            ni.append(jnp.max(jnp.where(sel, ci, -1), axis=1, keepdims=True))
            nv.append(m)
            cv = jnp.where(sel, -jnp.inf, cv)
        run_v = jnp.concatenate(nv, axis=1)
        run_i = jnp.concatenate(ni, axis=1)
    tv_ref[...] = run_v
    ti_ref[...] = run_i


def _topk_call(e, et):
    grid = (N_PAD // RB,)
    return pl.pallas_call(
        _topk_body,
        grid=grid,
        in_specs=[
            pl.BlockSpec((RB, H2), lambda i: (i, 0)),
            pl.BlockSpec((H2, N_PAD), lambda i: (0, 0)),
        ],
        out_specs=[
            pl.BlockSpec((RB, K), lambda i: (i, 0)),
            pl.BlockSpec((RB, K), lambda i: (i, 0)),
        ],
        out_shape=[
            jax.ShapeDtypeStruct((N_PAD, K), jnp.float32),
            jax.ShapeDtypeStruct((N_PAD, K), jnp.int32),
        ],
        interpret=_INTERPRET,
    )(e, et)


BLK = 64
EP_A = 161792
EP_B = 81920
NBLK_A = EP_A // 16 // BLK
NBLK_B = EP_B // 16 // BLK
NPT = N_PAD // 16
FCH = 32


def _edge_call(msg4, sT, selfp_t, pk, zrow, zd, nblk, phase, agg_den=None):
    mesh = plsc.VectorSubcoreMesh(core_axis_name="c", subcore_axis_name="s")
    if phase == "a":
        out_type = (jax.ShapeDtypeStruct((T, N_PAD, C), jnp.float32),
                    jax.ShapeDtypeStruct((T, N_PAD), jnp.float32))
    else:
        out_type = jax.ShapeDtypeStruct((T, N_PAD, C), jnp.float32)

    @functools.partial(
        pl.kernel,
        mesh=mesh,
        compiler_params=pltpu.CompilerParams(needs_layout_passes=False),
        out_type=out_type,
        scratch_types=[
            pltpu.VMEM_SHARED((N_PAD, C), jnp.float32),
            pltpu.VMEM_SHARED((N_PAD,), jnp.float32),
            pltpu.VMEM((BLK, C), jnp.float32),
            pltpu.VMEM((BLK, C), jnp.float32),
            pltpu.VMEM((N_PAD,), jnp.float32),
            pltpu.VMEM((N_PAD,), jnp.float32),
            pltpu.VMEM((3 * BLK,), jnp.int32),
            pltpu.VMEM((3 * BLK,), jnp.int32),
            pltpu.VMEM((BLK,), jnp.int32),
            pltpu.VMEM((BLK,), jnp.int32),
            pltpu.VMEM((BLK,), jnp.float32),
            pltpu.VMEM((BLK,), jnp.float32),
            pltpu.VMEM((BLK,), jnp.int32),
            pltpu.VMEM((FCH,), jnp.float32),
            pltpu.SemaphoreType.DMA,
            pltpu.SemaphoreType.DMA,
            pltpu.SemaphoreType.DMA,
        ],
    )
    def k(*refs):
        if phase == "a":
            (msg4_h, sT_h, selfp_h, pk_h, zrow_h, zd_h, aggO_h, denO_h,
             agg_sh, den_sh, rows0, rows1, ssrc_t, sdst_t, pkb0, pkb1,
             dst0, dst1, ex_b, wt_b, mix_b, inv_b, gsem, ssem0, ssem1) = refs
        else:
            (msg4_h, sT_h, selfp_h, pk_h, aggA_h, denA_h, out_h,
             agg_sh, den_sh, rows0, rows1, ssrc_t, sdst_t, pkb0, pkb1,
             dst0, dst1, ex_b, wt_b, mix_b, inv_b, gsem, ssem0, ssem1) = refs
        cc = lax.axis_index("c")
        ss = lax.axis_index("s")
        iota16 = lax.broadcasted_iota(jnp.int32, (16,), 0)
        z16 = jnp.zeros((16,), jnp.int32)
        n0 = ss * NPT
        blk0 = ss * nblk
        bufs = ((rows0, dst0, pkb0, ssem0), (rows1, dst1, pkb1, ssem1))
        for tp in range(2):
            t = 2 * cc + tp
            if phase == "a":
                pltpu.sync_copy(zrow_h, agg_sh.at[pl.ds(n0, NPT)])
                pltpu.sync_copy(zd_h, den_sh.at[pl.ds(n0, NPT)])
            else:
                pltpu.sync_copy(aggA_h.at[t, pl.ds(n0, NPT)],
                                agg_sh.at[pl.ds(n0, NPT)])
                pltpu.sync_copy(denA_h.at[t, pl.ds(n0, NPT)],
                                den_sh.at[pl.ds(n0, NPT)])
            pltpu.sync_copy(sT_h.at[t], ssrc_t)
            pltpu.sync_copy(sT_h.at[t + T], sdst_t)
            plsc.subcore_barrier()

            def pair_body(ob, carry):
                for p in range(2):
                    rowsP, dstP, pkbP, ssemP = bufs[p]
                    bb = ob * 2 + p

                    @pl.when(ob > 0)
                    def _drain():
                        pltpu.make_async_copy(
                            rowsP, agg_sh.at[dstP], ssemP).wait()

                    pltpu.sync_copy(
                        pk_h.at[pl.ds((blk0 + bb) * (3 * BLK), 3 * BLK)],
                        pkbP)
                    for j in range(BLK // 16):
                        sl = pl.ds(j * 16, 16)
                        sv = pkbP[sl]
                        dv = pkbP[pl.ds(BLK + j * 16, 16)]
                        eav16 = plsc.bitcast(
                            pkbP[pl.ds(2 * BLK + j * 16, 16)], jnp.float32)
                        a = (plsc.load_gather(ssrc_t, [sv])
                             + plsc.load_gather(sdst_t, [dv]))
                        a = jnp.maximum(a, 0.2 * a)
                        ex = jnp.exp(a)
                        ex_b[sl] = ex
                        wt_b[sl] = ex * eav16
                        dstP[sl] = dv
                        mix_b[sl] = sv * T + t
                    pltpu.sync_copy(ex_b, den_sh.at[dstP], add=True)
                    pltpu.async_copy(msg4_h.at[mix_b], rowsP, gsem).wait()

                    def e_body(ie, c2):
                        w16 = plsc.load_gather(wt_b, [z16 + ie])
                        rid = z16 + ie
                        for kk in range(C // 16):
                            col = iota16 + (kk * 16)
                            v = plsc.load_gather(rowsP, [rid, col])
                            plsc.store_scatter(rowsP, [rid, col], v * w16)
                        return c2
                    lax.fori_loop(0, BLK, e_body, 0)
                    pltpu.async_copy(rowsP, agg_sh.at[dstP], ssemP, add=True)
                return carry

            lax.fori_loop(0, nblk // 2, pair_body, 0)
            pltpu.make_async_copy(rows0, agg_sh.at[dst0], ssem0).wait()
            pltpu.make_async_copy(rows1, agg_sh.at[dst1], ssem1).wait()
            plsc.subcore_barrier()
            if phase == "a":
                pltpu.sync_copy(agg_sh.at[pl.ds(n0, NPT)],
                                aggO_h.at[t, pl.ds(n0, NPT)])
                pltpu.sync_copy(den_sh.at[pl.ds(n0, NPT)],
                                denO_h.at[t, pl.ds(n0, NPT)])
            else:
                for ck in range(NPT // FCH):
                    nb = n0 + ck * FCH
                    pltpu.sync_copy(den_sh.at[pl.ds(nb, FCH)], inv_b)
                    for j in range(FCH // 16):
                        sl = pl.ds(j * 16, 16)
                        inv_b[sl] = 1.0 / (inv_b[sl] + 1e-16)
                    pltpu.sync_copy(agg_sh.at[pl.ds(nb, FCH)],
                                    rows0.at[pl.ds(0, FCH)])
                    pltpu.sync_copy(selfp_h.at[t, pl.ds(nb, FCH)],
                                    rows0.at[pl.ds(FCH, FCH)])

                    def n_body(jn, c2):
                        w16 = plsc.load_gather(inv_b, [z16 + jn])
                        for kk in range(C // 16):
                            col = iota16 + (kk * 16)
                            v = plsc.load_gather(rows0, [z16 + jn, col])
                            sv_ = plsc.load_gather(rows0,
                                                   [z16 + (FCH + jn), col])
                            plsc.store_scatter(rows1, [z16 + jn, col],
                                               v * w16 + sv_)
                        return c2
                    lax.fori_loop(0, FCH, n_body, 0)
                    pltpu.sync_copy(rows1.at[pl.ds(0, FCH)],
                                    out_h.at[t, pl.ds(nb, FCH)])
            plsc.subcore_barrier()

    if phase == "a":
        return k(msg4, sT, selfp_t, pk, zrow, zd)
    return k(msg4, sT, selfp_t, pk, agg_den[0], agg_den[1])


def kernel(x, edge_index, edge_attr, node_proj_W, mix_logit, conv_W, conv_b,
           lin_msg_W, lin_msg_b, lin_self_W, lin_self_b,
           att_src_W, att_src_b, att_dst_W, att_dst_b):
    f32 = jnp.float32
    x_flat = x.reshape(N, TC_FLAT)
    x_flat = jnp.pad(x_flat, ((0, N_PAD - N), (0, 0)))

    eyeT = jnp.eye(T, dtype=f32)
    blocks = []
    for t_in in range(T):
        row = []
        for t_out in range(T):
            k = t_in - t_out + 1
            if 0 <= k <= 2:
                row.append(conv_W[:, :, k].T)
            else:
                row.append(jnp.zeros((C, C), f32))
        blocks.append(jnp.concatenate(row, axis=1))
    wconv = jnp.concatenate(blocks, axis=0)
    wm_bd = jnp.kron(eyeT, lin_msg_W.T)
    wmsg = wconv @ wm_bd
    bmsg_t = conv_b @ lin_msg_W.T + lin_msg_b
    bmsg = jnp.tile(bmsg_t, (T,))[None, :]
    wself = jnp.kron(eyeT, lin_self_W.T)
    bself = jnp.tile(lin_self_b, (T,))[None, :]
    wproj = jnp.tile(node_proj_W.T, (T, 1)) / T
    sv = jnp.zeros((TC_FLAT, 2 * T), f32)
    for t in range(T):
        sv = sv.at[t * C:(t + 1) * C, t].set(att_src_W[0])
        sv = sv.at[t * C:(t + 1) * C, T + t].set(att_dst_W[0])
    sb = jnp.concatenate([jnp.tile(att_src_b, (T,)),
                          jnp.tile(att_dst_b, (T,))])[None, :]

    def _pack(s_, d_, a_):
        return jnp.concatenate(
            [s_.reshape(-1, BLK), d_.reshape(-1, BLK),
             jax.lax.bitcast_convert_type(a_, jnp.int32).reshape(-1, BLK)],
            axis=1).reshape(-1)

    msg_flat, selfp_t, sT, e, et = _dense_call(
        x_flat, wmsg, bmsg, wself, bself, wproj, sv, sb)
    msg4 = msg_flat.reshape(N_PAD * T, C)
    zrow = jnp.zeros((NPT, C), f32)
    zd = jnp.zeros((NPT,), f32)
    alpha = jax.nn.sigmoid(mix_logit)

    e_fixed = edge_index.shape[1]
    npad_a = EP_A - e_fixed
    pad_dst_a = N + (jnp.arange(npad_a, dtype=jnp.int32) % (N_PAD - N))
    pk_a = _pack(
        jnp.concatenate([edge_index[0].astype(jnp.int32),
                         jnp.zeros((npad_a,), jnp.int32)]),
        jnp.concatenate([edge_index[1].astype(jnp.int32), pad_dst_a]),
        jnp.concatenate([edge_attr[:, 0] * (1.0 - alpha),
                         jnp.zeros((npad_a,), f32)]))
    aggA, denA = _edge_call(msg4, sT, selfp_t, pk_a, zrow, zd, NBLK_A, "a")

    tv_p, ti_p = _topk_call(e, et)
    tv = tv_p[:N]
    ti = ti_p[:N]

    npad_b = EP_B - N * K
    src_dyn = jnp.arange(N * K, dtype=jnp.int32) // K
    pad_dst_b = N + (jnp.arange(npad_b, dtype=jnp.int32) % (N_PAD - N))
    pk_b = _pack(
        jnp.concatenate([src_dyn, jnp.zeros((npad_b,), jnp.int32)]),
        jnp.concatenate([ti.reshape(-1), pad_dst_b]),
        jnp.concatenate([tv.reshape(-1) * alpha, jnp.zeros((npad_b,), f32)]))
    out_t = _edge_call(msg4, sT, selfp_t, pk_b, None, None, NBLK_B, "b",
                       agg_den=(aggA, denA))
    return out_t.transpose(1, 0, 2)[:N]
